# baseline (speedup 1.0000x reference)
import functools

import jax
import jax.numpy as jnp
from jax import lax
from jax.experimental import pallas as pl
from jax.experimental.pallas import tpu as pltpu
from jax.experimental.pallas import tpu_sc as plsc

NUM_ROWS = 1_000_000
DIM = 64
NA, NT = 4096, 200
BATCH = NA * NT
NC, NS = 2, 16
NW = NC * NS
CHUNK = 128
NCH = BATCH // NW // CHUNK
OPC = CHUNK // 2
PFR = CHUNK // 8
NBUF = 4
NOB = 2

_mesh = plsc.VectorSubcoreMesh(core_axis_name="c", subcore_axis_name="s")


@functools.partial(
    pl.kernel,
    mesh=_mesh,
    out_type=jax.ShapeDtypeStruct((BATCH // 2, 2 * DIM), jnp.float32),
    scratch_types=[
        pltpu.VMEM((NBUF, CHUNK), jnp.int32),
        *[pltpu.VMEM((PFR, CHUNK), jnp.float32) for _ in range(NBUF)],
        *[pltpu.VMEM((CHUNK, 2 * DIM), jnp.float32) for _ in range(NBUF)],
        *[pltpu.VMEM((OPC, 2 * DIM), jnp.float32) for _ in range(NOB)],
        *[pltpu.SemaphoreType.DMA for _ in range(2 * NBUF + NOB)],
    ],
)
def _gather(xs_hbm, pf_hbm, wp_hbm, out_hbm, jbuf, *bufs):
    pfs = bufs[0:NBUF]
    rows = bufs[NBUF:2 * NBUF]
    obs = bufs[2 * NBUF:2 * NBUF + NOB]
    sems = bufs[2 * NBUF + NOB:]
    sjs = sems[0:NBUF]
    sgs = sems[NBUF:2 * NBUF]
    sos = sems[2 * NBUF:]

    wid = lax.axis_index("s") * NC + lax.axis_index("c")
    xbase = wid * NCH

    def fire(j, p):
        pltpu.make_async_copy(
            xs_hbm.at[xbase + j], jbuf.at[p], sjs[p]
        ).start()
        pltpu.make_async_copy(
            pf_hbm.at[pl.ds((xbase + j) * PFR, PFR), :], pfs[p], sjs[p]
        ).start()

    def start_gather(j, p):
        pltpu.make_async_copy(
            xs_hbm.at[xbase + j], jbuf.at[p], sjs[p]
        ).wait()
        pltpu.make_async_copy(
            pf_hbm.at[pl.ds((xbase + j) * PFR, PFR), :], pfs[p], sjs[p]
        ).wait()
        pltpu.make_async_copy(wp_hbm.at[jbuf.at[p]], rows[p], sgs[p]).start()

    for p in range(NBUF):
        fire(p, p)
    for p in range(NBUF - 1):
        start_gather(p, p)

    def body(i, carry):
        for p in range(NBUF):
            j = NBUF * i + p
            q = p % NOB

            pltpu.make_async_copy(wp_hbm.at[jbuf.at[p]], rows[p], sgs[p]).wait()

            @pl.when(j + NBUF - 1 < NCH)
            def _():
                start_gather(j + NBUF - 1, (p + NBUF - 1) % NBUF)

            @pl.when(j >= NOB)
            def _():
                pltpu.make_async_copy(
                    obs[q], out_hbm.at[pl.ds(0, OPC), :], sos[q]
                ).wait()

            def pack(g, carry2):
                for h in range(8):
                    r = 8 * g + h
                    pv = pfs[p][g, pl.ds(16 * h, 16)]
                    for cc in range(0, DIM, 16):
                        lo = rows[p][r, pl.ds(cc, 16)]
                        hi = rows[p][r, pl.ds(DIM + cc, 16)]
                        obs[q][4 * g + h // 2, pl.ds((h % 2) * DIM + cc, 16)] = (
                            lo + (hi - lo) * pv
                        )
                return carry2

            lax.fori_loop(0, CHUNK // 8, pack, 0)

            @pl.when(j + NBUF < NCH)
            def _():
                fire(j + NBUF, p)

            pltpu.make_async_copy(
                obs[q],
                out_hbm.at[pl.ds((xbase + j) * OPC, OPC), :],
                sos[q],
            ).start()

        return carry

    lax.fori_loop(0, NCH // NBUF, body, 0)

    for q in range(NOB):
        j = NCH - NOB + q
        pltpu.make_async_copy(
            obs[(j % NBUF) % NOB],
            out_hbm.at[pl.ds((xbase + j) * OPC, OPC), :],
            sos[(j % NBUF) % NOB],
        ).wait()


def kernel(x, weight):
    xi = x.astype(jnp.int32)
    xs2 = (xi >> 1).reshape(BATCH // CHUNK, CHUNK)
    pf = jnp.broadcast_to(
        (xi & 1).astype(jnp.float32).reshape(BATCH, 1), (BATCH, 16)
    ).reshape(BATCH * 16 // CHUNK, CHUNK)
    wp = weight.reshape(NUM_ROWS // 2, 2 * DIM)
    out = _gather(xs2, pf, wp)
    return out.reshape(NA, NT, DIM)


# speedup vs baseline: 1.2058x; 1.2058x over previous
import functools

import jax
import jax.numpy as jnp
from jax import lax
from jax.experimental import pallas as pl
from jax.experimental.pallas import tpu as pltpu
from jax.experimental.pallas import tpu_sc as plsc

NUM_ROWS = 1_000_000
DIM = 64
NA, NT = 4096, 200
BATCH = NA * NT
NC, NS = 2, 16
NW = NC * NS
CHUNK = 128
NCH = BATCH // NW // CHUNK
OPC = CHUNK // 2
PFR = CHUNK // 8
NBUF = 4
NOB = 2

_mesh = plsc.VectorSubcoreMesh(core_axis_name="c", subcore_axis_name="s")


@functools.partial(
    pl.kernel,
    mesh=_mesh,
    out_type=jax.ShapeDtypeStruct((BATCH // 2, 2 * DIM), jnp.float32),
    scratch_types=[
        pltpu.VMEM((NBUF, CHUNK), jnp.int32),
        *[pltpu.VMEM((PFR, CHUNK), jnp.float32) for _ in range(NBUF)],
        *[pltpu.VMEM((CHUNK, 2 * DIM), jnp.float32) for _ in range(NBUF)],
        *[pltpu.VMEM((OPC, 2 * DIM), jnp.float32) for _ in range(NOB)],
        *[pltpu.SemaphoreType.DMA for _ in range(2 * NBUF + NOB)],
    ],
)
def _gather(xs_hbm, pf_hbm, wp_hbm, out_hbm, jbuf, *bufs):
    pfs = bufs[0:NBUF]
    rows = bufs[NBUF:2 * NBUF]
    obs = bufs[2 * NBUF:2 * NBUF + NOB]
    sems = bufs[2 * NBUF + NOB:]
    sjs = sems[0:NBUF]
    sgs = sems[NBUF:2 * NBUF]
    sos = sems[2 * NBUF:]

    wid = lax.axis_index("s") * NC + lax.axis_index("c")
    xbase = wid * NCH

    def fire(j, p):
        pltpu.make_async_copy(
            xs_hbm.at[xbase + j], jbuf.at[p], sjs[p]
        ).start()
        pltpu.make_async_copy(
            pf_hbm.at[pl.ds((xbase + j) * PFR, PFR), :], pfs[p], sjs[p]
        ).start()

    def start_gather(j, p):
        pltpu.make_async_copy(
            xs_hbm.at[xbase + j], jbuf.at[p], sjs[p]
        ).wait()
        pltpu.make_async_copy(
            pf_hbm.at[pl.ds((xbase + j) * PFR, PFR), :], pfs[p], sjs[p]
        ).wait()
        pltpu.make_async_copy(wp_hbm.at[jbuf.at[p]], rows[p], sgs[p]).start()

    for p in range(NBUF):
        fire(p, p)
    for p in range(NBUF - 1):
        start_gather(p, p)

    def body(i, carry):
        for p in range(NBUF):
            j = NBUF * i + p
            q = p % NOB

            pltpu.make_async_copy(wp_hbm.at[jbuf.at[p]], rows[p], sgs[p]).wait()

            @pl.when(j + NBUF - 1 < NCH)
            def _():
                start_gather(j + NBUF - 1, (p + NBUF - 1) % NBUF)

            @pl.when(j >= NOB)
            def _():
                pltpu.make_async_copy(
                    obs[q], out_hbm.at[pl.ds(0, OPC), :], sos[q]
                ).wait()

            @plsc.parallel_loop(0, CHUNK // 8, unroll=2)
            def pack(g):
                for h in range(8):
                    r = 8 * g + h
                    pv = pfs[p][g, pl.ds(16 * h, 16)]
                    for cc in range(0, DIM, 16):
                        lo = rows[p][r, pl.ds(cc, 16)]
                        hi = rows[p][r, pl.ds(DIM + cc, 16)]
                        obs[q][4 * g + h // 2, pl.ds((h % 2) * DIM + cc, 16)] = (
                            lo + (hi - lo) * pv
                        )

            @pl.when(j + NBUF < NCH)
            def _():
                fire(j + NBUF, p)

            pltpu.make_async_copy(
                obs[q],
                out_hbm.at[pl.ds((xbase + j) * OPC, OPC), :],
                sos[q],
            ).start()

        return carry

    lax.fori_loop(0, NCH // NBUF, body, 0)

    for q in range(NOB):
        j = NCH - NOB + q
        pltpu.make_async_copy(
            obs[(j % NBUF) % NOB],
            out_hbm.at[pl.ds((xbase + j) * OPC, OPC), :],
            sos[(j % NBUF) % NOB],
        ).wait()


def kernel(x, weight):
    xi = x.astype(jnp.int32)
    xs2 = (xi >> 1).reshape(BATCH // CHUNK, CHUNK)
    pf = jnp.broadcast_to(
        (xi & 1).astype(jnp.float32).reshape(BATCH, 1), (BATCH, 16)
    ).reshape(BATCH * 16 // CHUNK, CHUNK)
    wp = weight.reshape(NUM_ROWS // 2, 2 * DIM)
    out = _gather(xs2, pf, wp)
    return out.reshape(NA, NT, DIM)


# speedup vs baseline: 1.6680x; 1.3833x over previous
import functools

import jax
import jax.numpy as jnp
from jax import lax
from jax.experimental import pallas as pl
from jax.experimental.pallas import tpu as pltpu
from jax.experimental.pallas import tpu_sc as plsc

NUM_ROWS = 1_000_000
DIM = 64
NA, NT = 4096, 200
BATCH = NA * NT
NC, NS = 2, 16
NW = NC * NS
BPW = BATCH // NW
CHUNK = 128
NCH = BPW // CHUNK
NBUF = 8

_mesh = plsc.VectorSubcoreMesh(core_axis_name="c", subcore_axis_name="s")


@functools.partial(
    pl.kernel,
    mesh=_mesh,
    out_type=jax.ShapeDtypeStruct((BATCH, DIM), jnp.float32),
    compiler_params=pltpu.CompilerParams(use_tc_tiling_on_sc=False),
    scratch_types=[
        pltpu.VMEM((BPW,), jnp.int32),
        *[pltpu.VMEM((CHUNK, DIM), jnp.float32) for _ in range(NBUF)],
        pltpu.SemaphoreType.DMA,
        *[pltpu.SemaphoreType.DMA for _ in range(NBUF)],
        *[pltpu.SemaphoreType.DMA for _ in range(NBUF)],
    ],
)
def _gather(x_hbm, w_hbm, out_hbm, idx_v, *bufs):
    rows = bufs[0:NBUF]
    semi = bufs[NBUF]
    sgs = bufs[NBUF + 1:2 * NBUF + 1]
    sos = bufs[2 * NBUF + 1:]

    wid = lax.axis_index("s") * NC + lax.axis_index("c")
    base = wid * BPW

    pltpu.make_async_copy(x_hbm.at[pl.ds(base, BPW)], idx_v, semi).start()
    pltpu.make_async_copy(x_hbm.at[pl.ds(base, BPW)], idx_v, semi).wait()

    def start_gather(j, p):
        pltpu.make_async_copy(
            w_hbm.at[idx_v.at[pl.ds(j * CHUNK, CHUNK)]], rows[p], sgs[p]
        ).start()

    for p in range(NBUF - 2):
        start_gather(p, p)

    def body(i, carry):
        for p in range(NBUF):
            j = NBUF * i + p

            pltpu.make_async_copy(
                w_hbm.at[idx_v.at[pl.ds(j * CHUNK, CHUNK)]], rows[p], sgs[p]
            ).wait()
            pltpu.make_async_copy(
                rows[p],
                out_hbm.at[pl.ds(base + j * CHUNK, CHUNK), :],
                sos[p],
            ).start()

            pw = (p + NBUF - 2) % NBUF

            @pl.when(j >= 2)
            def _():
                pltpu.make_async_copy(
                    rows[pw], out_hbm.at[pl.ds(base, CHUNK), :], sos[pw]
                ).wait()

            @pl.when(j + NBUF - 2 < NCH)
            def _():
                start_gather(j + NBUF - 2, pw)

        return carry

    lax.fori_loop(0, NCH // NBUF, body, 0)

    for j in (NCH - 2, NCH - 1):
        p = j % NBUF
        pltpu.make_async_copy(
            rows[p], out_hbm.at[pl.ds(base, CHUNK), :], sos[p]
        ).wait()


def kernel(x, weight):
    x1 = x.reshape(BATCH).astype(jnp.int32)
    out = _gather(x1, weight)
    return out.reshape(NA, NT, DIM)


# speedup vs baseline: 1.6696x; 1.0009x over previous
import functools

import jax
import jax.numpy as jnp
from jax import lax
from jax.experimental import pallas as pl
from jax.experimental.pallas import tpu as pltpu
from jax.experimental.pallas import tpu_sc as plsc

NUM_ROWS = 1_000_000
DIM = 64
NA, NT = 4096, 200
BATCH = NA * NT
NC, NS = 2, 16
NW = NC * NS
BPW = BATCH // NW
CHUNK = 128
NCH = BPW // CHUNK
NBUF = 8

_mesh = plsc.VectorSubcoreMesh(core_axis_name="c", subcore_axis_name="s")


@functools.partial(
    pl.kernel,
    mesh=_mesh,
    out_type=jax.ShapeDtypeStruct((BATCH, DIM), jnp.float32),
    compiler_params=pltpu.CompilerParams(
        use_tc_tiling_on_sc=False, skip_device_barrier=True
    ),
    scratch_types=[
        pltpu.VMEM((BPW,), jnp.int32),
        *[pltpu.VMEM((CHUNK, DIM), jnp.float32) for _ in range(NBUF)],
        pltpu.SemaphoreType.DMA,
        *[pltpu.SemaphoreType.DMA for _ in range(NBUF)],
        *[pltpu.SemaphoreType.DMA for _ in range(NBUF)],
    ],
)
def _gather(x_hbm, w_hbm, out_hbm, idx_v, *bufs):
    rows = bufs[0:NBUF]
    semi = bufs[NBUF]
    sgs = bufs[NBUF + 1:2 * NBUF + 1]
    sos = bufs[2 * NBUF + 1:]

    wid = lax.axis_index("s") * NC + lax.axis_index("c")
    base = wid * BPW

    pltpu.make_async_copy(x_hbm.at[pl.ds(base, BPW)], idx_v, semi).start()
    pltpu.make_async_copy(x_hbm.at[pl.ds(base, BPW)], idx_v, semi).wait()

    def start_gather(j, p):
        pltpu.make_async_copy(
            w_hbm.at[idx_v.at[pl.ds(j * CHUNK, CHUNK)]], rows[p], sgs[p]
        ).start()

    for p in range(NBUF - 2):
        start_gather(p, p)

    def body(i, carry):
        for p in range(NBUF):
            j = NBUF * i + p

            pltpu.make_async_copy(
                w_hbm.at[idx_v.at[pl.ds(j * CHUNK, CHUNK)]], rows[p], sgs[p]
            ).wait()
            pltpu.make_async_copy(
                rows[p],
                out_hbm.at[pl.ds(base + j * CHUNK, CHUNK), :],
                sos[p],
            ).start()

            pw = (p + NBUF - 2) % NBUF

            @pl.when(j >= 2)
            def _():
                pltpu.make_async_copy(
                    rows[pw], out_hbm.at[pl.ds(base, CHUNK), :], sos[pw]
                ).wait()

            @pl.when(j + NBUF - 2 < NCH)
            def _():
                start_gather(j + NBUF - 2, pw)

        return carry

    lax.fori_loop(0, NCH // NBUF, body, 0)

    for j in (NCH - 2, NCH - 1):
        p = j % NBUF
        pltpu.make_async_copy(
            rows[p], out_hbm.at[pl.ds(base, CHUNK), :], sos[p]
        ).wait()


def kernel(x, weight):
    x1 = x.reshape(BATCH).astype(jnp.int32)
    out = _gather(x1, weight)
    return out.reshape(NA, NT, DIM)
